# baseline (speedup 1.0000x reference)
import functools

import jax
import jax.numpy as jnp
from jax import lax
from jax.experimental import pallas as pl
from jax.experimental.pallas import tpu as pltpu
from jax.experimental.pallas import tpu_sc as plsc

N = 10000
NP = 10240
E = 320000
H = 128
NG = 16
NLAYERS = 4

RT = 256
NT_N = NP // RT
NT_E = E // RT

NC, NS = 2, 16
NW = NC * NS
EPW = E // NW
CH = 40
NCH = EPW // CH
SBC = 50
NSB = NCH // SBC
RPS = NP // NS
CHD = 80
NCHD = EPW // CHD

_F32 = jnp.float32
_BF16 = jnp.bfloat16

_PERM = tuple(
    32 * (i // 32) + (2 * (i % 32) if i % 32 < 16 else 2 * (i % 32 - 16) + 1)
    for i in range(H))


def _silu(x):
    return x / (1.0 + jnp.exp(-x))


def _nodeproj_body(x_ref, w1, b1, w2, b2, ws, wd, h_ref, a_ref, b_ref):
    u = _silu(jnp.dot(x_ref[...], w1[...], preferred_element_type=_F32) + b1[...])
    h = jnp.dot(u, w2[...], preferred_element_type=_F32) + b2[...]
    h_ref[...] = h
    a_ref[...] = jnp.dot(h, ws[...], preferred_element_type=_F32)
    b_ref[...] = jnp.dot(h, wd[...], preferred_element_type=_F32)


def _node_proj(xp, w1, b1, w2, b2, ws, wd):
    wspec = pl.BlockSpec((H, H), lambda i: (0, 0))
    bspec = pl.BlockSpec((1, H), lambda i: (0, 0))
    rspec = pl.BlockSpec((RT, H), lambda i: (i, 0))
    return pl.pallas_call(
        _nodeproj_body,
        grid=(NT_N,),
        in_specs=[rspec, wspec, bspec, wspec, bspec, wspec, wspec],
        out_specs=[rspec, rspec, rspec],
        out_shape=[jax.ShapeDtypeStruct((NP, H), _F32)] * 3,
    )(xp, w1, b1, w2, b2, ws, wd)


def _edgec_body(ea_ref, we1, be1, m_ref, c_ref, *out_refs):
    u = _silu(jnp.dot(ea_ref[...], we1[...], preferred_element_type=_F32) + be1[...])
    for l in range(len(out_refs)):
        out_refs[l][...] = (
            jnp.dot(u, m_ref[l], preferred_element_type=_F32) + c_ref[l])


def _edge_terms(edge_attr, we1, be1, ms, cs):
    nl = ms.shape[0]
    espec = pl.BlockSpec((RT, 16), lambda i: (i, 0))
    rspec = pl.BlockSpec((RT, H), lambda i: (i, 0))
    return pl.pallas_call(
        _edgec_body,
        grid=(NT_E,),
        in_specs=[
            espec,
            pl.BlockSpec((16, H), lambda i: (0, 0)),
            pl.BlockSpec((1, H), lambda i: (0, 0)),
            pl.BlockSpec((nl, H, H), lambda i: (0, 0, 0)),
            pl.BlockSpec((nl, 1, H), lambda i: (0, 0, 0)),
        ],
        out_specs=[rspec] * nl,
        out_shape=[jax.ShapeDtypeStruct((E, H), _F32)] * nl,
    )(edge_attr, we1, be1, ms, cs)


def _update_body(h_ref, s_ref, degb_ref, w2m, b2m, wu1a, wu1b, bu1, wu2, bu2,
                 lng, lnb, wsn, wdn, hn_ref, an_ref, bn_ref):
    s = s_ref[0] + s_ref[1]
    agg = jnp.dot(s, w2m[...], preferred_element_type=_F32) + degb_ref[...] * b2m[...]
    h = h_ref[...]
    z = _silu(jnp.dot(h, wu1a[...], preferred_element_type=_F32)
              + jnp.dot(agg, wu1b[...], preferred_element_type=_F32) + bu1[...])
    r = h + jnp.dot(z, wu2[...], preferred_element_type=_F32) + bu2[...]
    m = jnp.mean(r, axis=1, keepdims=True)
    d = r - m
    v = jnp.mean(d * d, axis=1, keepdims=True)
    hn = d * lax.rsqrt(v + 1e-5) * lng[...] + lnb[...]
    hn_ref[...] = hn
    an_ref[...] = jnp.dot(hn, wsn[...], preferred_element_type=_F32)
    bn_ref[...] = jnp.dot(hn, wdn[...], preferred_element_type=_F32)


def _node_update(h, s2, degb, w2m, b2m, wu1a, wu1b, bu1, wu2, bu2, lng, lnb,
                 wsn, wdn):
    wspec = pl.BlockSpec((H, H), lambda i: (0, 0))
    bspec = pl.BlockSpec((1, H), lambda i: (0, 0))
    rspec = pl.BlockSpec((RT, H), lambda i: (i, 0))
    return pl.pallas_call(
        _update_body,
        grid=(NT_N,),
        in_specs=[
            rspec,
            pl.BlockSpec((2, RT, H), lambda i: (0, i, 0)),
            rspec, wspec, bspec, wspec, wspec, bspec, wspec, bspec,
            bspec, bspec, wspec, wspec,
        ],
        out_specs=[rspec, rspec, rspec],
        out_shape=[jax.ShapeDtypeStruct((NP, H), _F32)] * 3,
    )(h, s2, degb, w2m, b2m, wu1a, wu1b, bu1, wu2, bu2, lng, lnb, wsn, wdn)


def _pool_body(h_ref, bi_ref, bf_ref, wo1, bo1, wo2, bo2,
               lat_ref, gr_ref, sum_acc, cnt_acc, max_acc):
    i = pl.program_id(0)

    @pl.when(i == 0)
    def _():
        sum_acc[...] = jnp.zeros_like(sum_acc)
        cnt_acc[...] = jnp.zeros_like(cnt_acc)
        max_acc[...] = jnp.full_like(max_acc, -1e30)

    h = h_ref[...]
    br = bi_ref[0]
    onehot = (lax.broadcasted_iota(jnp.int32, (NG, RT), 0) == br).astype(_F32)
    sum_acc[...] += jnp.dot(onehot, h, preferred_element_type=_F32)
    cnt_acc[...] += jnp.broadcast_to(
        jnp.sum(onehot, axis=1, keepdims=True), (NG, H))
    bf = bf_ref[...]
    for g in range(NG):
        masked = jnp.where(bf == float(g), h, -1e30)
        mg = jnp.max(masked, axis=0, keepdims=True)
        max_acc[g:g + 1, :] = jnp.maximum(max_acc[g:g + 1, :], mg)

    @pl.when(i == NT_N - 1)
    def _():
        cnt = cnt_acc[...]
        mean = sum_acc[...] / jnp.maximum(cnt, 1.0)
        mx = jnp.where(cnt > 0.0, max_acc[...], 0.0)
        gr = jnp.concatenate([mean, mx], axis=1)
        gr_ref[...] = gr
        z = _silu(jnp.dot(gr, wo1[...], preferred_element_type=_F32) + bo1[...])
        lat_ref[...] = jnp.dot(z, wo2[...], preferred_element_type=_F32) + bo2[...]


def _pool_out(h, batch3, bf, wo1, bo1, wo2, bo2):
    return pl.pallas_call(
        _pool_body,
        grid=(NT_N,),
        in_specs=[
            pl.BlockSpec((RT, H), lambda i: (i, 0)),
            pl.BlockSpec((1, 1, RT), lambda i: (i, 0, 0)),
            pl.BlockSpec((RT, H), lambda i: (i, 0)),
            pl.BlockSpec((2 * H, H), lambda i: (0, 0)),
            pl.BlockSpec((1, H), lambda i: (0, 0)),
            pl.BlockSpec((H, H), lambda i: (0, 0)),
            pl.BlockSpec((1, H), lambda i: (0, 0)),
        ],
        out_specs=[
            pl.BlockSpec((NG, H), lambda i: (0, 0)),
            pl.BlockSpec((NG, 2 * H), lambda i: (0, 0)),
        ],
        out_shape=[
            jax.ShapeDtypeStruct((NG, H), _F32),
            jax.ShapeDtypeStruct((NG, 2 * H), _F32),
        ],
        scratch_shapes=[
            pltpu.VMEM((NG, H), _F32),
            pltpu.VMEM((NG, H), _F32),
            pltpu.VMEM((NG, H), _F32),
        ],
    )(h, batch3, bf, wo1, bo1, wo2, bo2)


@functools.lru_cache(maxsize=None)
def _sc_mesh():
    return plsc.VectorSubcoreMesh(
        core_axis_name="c", subcore_axis_name="s",
        num_cores=NC, num_subcores=NS)


def _sc_edge_body(a_hbm, b_hbm, c_hbm, src_hbm, dst_hbm, out_hbm,
                  sidx, didx, didxb, abuf, bbuf, cbuf, s_sh,
                  gsem, ssem):
    cid = lax.axis_index("c")
    sid = lax.axis_index("s")
    wid = sid * NC + cid
    base = wid * EPW

    zv = jnp.zeros((16,), _F32)

    def zrow(e, _):
        for g in range(8):
            cbuf[0][e, pl.ds(g * 16, 16)] = zv
        return 0
    lax.fori_loop(0, CH, zrow, 0)
    for j in range(RPS // CH):
        pltpu.async_copy(cbuf[0], s_sh.at[pl.ds(sid * RPS + j * CH, CH)],
                         gsem[0])
    for j in range(RPS // CH):
        pltpu.make_async_copy(
            cbuf[0], s_sh.at[pl.ds(sid * RPS + j * CH, CH)], gsem[0]).wait()
    plsc.subcore_barrier()

    def fire(lc, p, sb, wait_scatter):
        if wait_scatter:
            pltpu.make_async_copy(abuf[p], s_sh.at[didxb[p]], ssem[p]).wait()
        t = sb * SBC + lc
        pltpu.async_copy(a_hbm.at[sidx.at[pl.ds(lc * CH, CH)]], abuf[p], gsem[p])
        pltpu.async_copy(b_hbm.at[didx.at[pl.ds(lc * CH, CH)]], bbuf[p], gsem[p])
        pltpu.async_copy(c_hbm.at[pl.ds(base + t * CH, CH)], cbuf[p], gsem[p])

    def process(lc, p, sb):
        for _ in range(3):
            pltpu.make_async_copy(
                c_hbm.at[pl.ds(base + sb * SBC * CH, CH)], cbuf[p],
                gsem[p]).wait()
        for o in (0, 16, 24):
            didxb[p][pl.ds(o, 16)] = didx[pl.ds(lc * CH + o, 16)]

        def ebody(e, _):
            for g in range(8):
                sl = pl.ds(g * 16, 16)
                v = abuf[p][e, sl] + bbuf[p][e, sl] + cbuf[p][e, sl]
                abuf[p][e, sl] = v / (1.0 + jnp.exp(-v))
            return 0
        lax.fori_loop(0, CH, ebody, 0)
        pltpu.async_copy(abuf[p], s_sh.at[didxb[p]], ssem[p], add=True)

    for sb in range(NSB):
        pltpu.sync_copy(src_hbm.at[pl.ds(base + sb * SBC * CH, SBC * CH)], sidx)
        pltpu.sync_copy(dst_hbm.at[pl.ds(base + sb * SBC * CH, SBC * CH)], didx)
        w = sb > 0
        fire(0, 0, sb, w)
        fire(1, 1, sb, w)

        def pair(j, _):
            lc = 2 * j
            process(lc, 0, sb)
            fire(lc + 2, 0, sb, True)
            process(lc + 1, 1, sb)
            fire(lc + 3, 1, sb, True)
            return 0
        lax.fori_loop(0, SBC // 2 - 1, pair, 0)
        process(SBC - 2, 0, sb)
        process(SBC - 1, 1, sb)

    pltpu.make_async_copy(abuf[0], s_sh.at[didxb[0]], ssem[0]).wait()
    pltpu.make_async_copy(abuf[1], s_sh.at[didxb[1]], ssem[1]).wait()

    plsc.subcore_barrier()
    pltpu.sync_copy(s_sh.at[pl.ds(sid * RPS, RPS)],
                    out_hbm.at[cid, pl.ds(sid * RPS, RPS)])


def _sc_edge(a, b, c, src, dst):
    return pl.kernel(
        _sc_edge_body,
        out_type=jax.ShapeDtypeStruct((NC, NP, H), _F32),
        mesh=_sc_mesh(),
        scratch_types=[
            pltpu.VMEM((SBC * CH,), jnp.int32),
            pltpu.VMEM((SBC * CH,), jnp.int32),
            [pltpu.VMEM((CH,), jnp.int32)] * 2,
            [pltpu.VMEM((CH, H), _F32)] * 2,
            [pltpu.VMEM((CH, H), _F32)] * 2,
            [pltpu.VMEM((CH, H), _F32)] * 2,
            pltpu.VMEM_SHARED((NP, H), _F32),
            [pltpu.SemaphoreType.DMA] * 2,
            [pltpu.SemaphoreType.DMA] * 2,
        ],
    )(a, b, c, src, dst)


def _sc_deg_body(dst_hbm, out_hbm, didx, obuf, s_sh):
    cid = lax.axis_index("c")
    sid = lax.axis_index("s")
    wid = sid * NC + cid
    base = wid * EPW

    zv = jnp.zeros((16,), _F32)

    def zrow(e, _):
        obuf[e, pl.ds(0, 16)] = zv
        return 0
    lax.fori_loop(0, CHD, zrow, 0)
    for j in range(RPS // CHD):
        pltpu.sync_copy(obuf, s_sh.at[pl.ds(sid * RPS + j * CHD, CHD)])
    plsc.subcore_barrier()

    ones = jnp.ones((16,), _F32)

    def orow(e, _):
        obuf[e, pl.ds(0, 16)] = ones
        return 0
    lax.fori_loop(0, CHD, orow, 0)

    def chunk(t, _):
        pltpu.sync_copy(dst_hbm.at[pl.ds(base + t * CHD, CHD)], didx)
        pltpu.sync_copy(obuf, s_sh.at[didx], add=True)
        return 0
    lax.fori_loop(0, NCHD, chunk, 0)

    plsc.subcore_barrier()
    pltpu.sync_copy(s_sh.at[pl.ds(sid * RPS, RPS)],
                    out_hbm.at[cid, pl.ds(sid * RPS, RPS)])


def _sc_deg(dst):
    return pl.kernel(
        _sc_deg_body,
        out_type=jax.ShapeDtypeStruct((NC, NP, 16), _F32),
        mesh=_sc_mesh(),
        scratch_types=[
            pltpu.VMEM((CHD,), jnp.int32),
            pltpu.VMEM((CHD, 16), _F32),
            pltpu.VMEM_SHARED((NP, 16), _F32),
        ],
    )(dst)


def kernel(x, edge_index, edge_attr, batch, params):
    src = edge_index[0]
    dst = edge_index[1]

    (wn1, bn1), (wn2, bn2) = params["node_proj"]
    (we1, be1), (we2, be2) = params["edge_proj"]
    layers = params["layers"]
    (wo1, bo1), (wo2, bo2) = params["out_proj"]

    ws, wd, ms, cs, w2m, b2m = [], [], [], [], [], []
    for lp in layers:
        (w1, b1), (w2, b2) = lp["msg"]
        ws.append(w1[0:H])
        wd.append(w1[H:2 * H])
        ms.append(we2 @ w1[2 * H:3 * H])
        cs.append((be2 @ w1[2 * H:3 * H] + b1)[None, :])
        w2m.append(w2)
        b2m.append(b2[None, :])
    ms = jnp.stack(ms)
    cs = jnp.stack(cs)

    xp = jnp.pad(x, ((0, NP - N), (0, 0)))
    batch_p = jnp.concatenate(
        [batch.astype(jnp.int32), jnp.full((NP - N,), NG, jnp.int32)])
    batch3 = batch_p.reshape(NT_N, 1, RT)
    bf = jnp.broadcast_to(batch_p[:, None].astype(_F32), (NP, H))

    degp = _sc_deg(dst)
    deg = degp[0, :, 0] + degp[1, :, 0]
    degb = jnp.broadcast_to(deg[:, None], (NP, H))

    h, a, b = _node_proj(xp, wn1, bn1[None, :], wn2, bn2[None, :], ws[0], wd[0])

    c0 = _edge_terms(edge_attr, we1, be1[None, :], ms[:1], cs[:1])
    c123 = _edge_terms(edge_attr, we1, be1[None, :], ms[1:], cs[1:])
    c_all = c0 + c123

    zw = jnp.zeros((H, H), _F32)
    for l in range(NLAYERS):
        lp = layers[l]
        (wu1, bu1), (wu2, bu2) = lp["upd"]
        lng, lnb = lp["ln"]
        s2 = _sc_edge(a, b, c_all[l], src, dst)
        wsn = ws[l + 1] if l + 1 < NLAYERS else zw
        wdn = wd[l + 1] if l + 1 < NLAYERS else zw
        h, a, b = _node_update(
            h, s2, degb, w2m[l], b2m[l],
            wu1[0:H], wu1[H:2 * H], bu1[None, :], wu2, bu2[None, :],
            lng[None, :], lnb[None, :], wsn, wdn)

    lat, gr = _pool_out(h, batch3, bf, wo1, bo1[None, :], wo2, bo2[None, :])
    return lat, gr


# speedup vs baseline: 1.3155x; 1.3155x over previous
import functools

import jax
import jax.numpy as jnp
from jax import lax
from jax.experimental import pallas as pl
from jax.experimental.pallas import tpu as pltpu
from jax.experimental.pallas import tpu_sc as plsc

N = 10000
NP = 10240
E = 320000
H = 128
NG = 16
NLAYERS = 4

RT = 256
NT_N = NP // RT
NT_E = E // RT

NC, NS = 2, 16
NW = NC * NS
EPW = E // NW
CH = 40
NCH = EPW // CH
SBC = 100
NSB = NCH // SBC
RPS = NP // NS
CHD = 80
NCHD = EPW // CHD

_F32 = jnp.float32
_BF16 = jnp.bfloat16

_PERM = tuple(
    32 * (i // 32) + (2 * (i % 32) if i % 32 < 16 else 2 * (i % 32 - 16) + 1)
    for i in range(H))


def _silu(x):
    return x / (1.0 + jnp.exp(-x))


def _nodeproj_body(x_ref, w1, b1, w2, b2, ws, wd, h_ref, a_ref, b_ref):
    u = _silu(jnp.dot(x_ref[...], w1[...], preferred_element_type=_F32) + b1[...])
    h = jnp.dot(u, w2[...], preferred_element_type=_F32) + b2[...]
    h_ref[...] = h
    a_ref[...] = jnp.dot(h, ws[...], preferred_element_type=_F32)
    b_ref[...] = jnp.dot(h, wd[...], preferred_element_type=_F32)


def _node_proj(xp, w1, b1, w2, b2, ws, wd):
    wspec = pl.BlockSpec((H, H), lambda i: (0, 0))
    bspec = pl.BlockSpec((1, H), lambda i: (0, 0))
    rspec = pl.BlockSpec((RT, H), lambda i: (i, 0))
    return pl.pallas_call(
        _nodeproj_body,
        grid=(NT_N,),
        in_specs=[rspec, wspec, bspec, wspec, bspec, wspec, wspec],
        out_specs=[rspec, rspec, rspec],
        out_shape=[jax.ShapeDtypeStruct((NP, H), _F32)] * 3,
    )(xp, w1, b1, w2, b2, ws, wd)


def _edgec_body(ea_ref, we1, be1, m_ref, c_ref, *out_refs):
    u = _silu(jnp.dot(ea_ref[...], we1[...], preferred_element_type=_F32) + be1[...])
    for l in range(len(out_refs)):
        out_refs[l][...] = (
            jnp.dot(u, m_ref[l], preferred_element_type=_F32) + c_ref[l])


def _edge_terms(edge_attr, we1, be1, ms, cs):
    nl = ms.shape[0]
    espec = pl.BlockSpec((RT, 16), lambda i: (i, 0))
    rspec = pl.BlockSpec((RT, H), lambda i: (i, 0))
    return pl.pallas_call(
        _edgec_body,
        grid=(NT_E,),
        in_specs=[
            espec,
            pl.BlockSpec((16, H), lambda i: (0, 0)),
            pl.BlockSpec((1, H), lambda i: (0, 0)),
            pl.BlockSpec((nl, H, H), lambda i: (0, 0, 0)),
            pl.BlockSpec((nl, 1, H), lambda i: (0, 0, 0)),
        ],
        out_specs=[rspec] * nl,
        out_shape=[jax.ShapeDtypeStruct((E, H), _F32)] * nl,
    )(edge_attr, we1, be1, ms, cs)


def _update_body(h_ref, s_ref, degb_ref, w2m, b2m, wu1a, wu1b, bu1, wu2, bu2,
                 lng, lnb, wsn, wdn, hn_ref, an_ref, bn_ref):
    s = s_ref[0] + s_ref[1]
    agg = jnp.dot(s, w2m[...], preferred_element_type=_F32) + degb_ref[...] * b2m[...]
    h = h_ref[...]
    z = _silu(jnp.dot(h, wu1a[...], preferred_element_type=_F32)
              + jnp.dot(agg, wu1b[...], preferred_element_type=_F32) + bu1[...])
    r = h + jnp.dot(z, wu2[...], preferred_element_type=_F32) + bu2[...]
    m = jnp.mean(r, axis=1, keepdims=True)
    d = r - m
    v = jnp.mean(d * d, axis=1, keepdims=True)
    hn = d * lax.rsqrt(v + 1e-5) * lng[...] + lnb[...]
    hn_ref[...] = hn
    an_ref[...] = jnp.dot(hn, wsn[...], preferred_element_type=_F32)
    bn_ref[...] = jnp.dot(hn, wdn[...], preferred_element_type=_F32)


def _node_update(h, s2, degb, w2m, b2m, wu1a, wu1b, bu1, wu2, bu2, lng, lnb,
                 wsn, wdn):
    wspec = pl.BlockSpec((H, H), lambda i: (0, 0))
    bspec = pl.BlockSpec((1, H), lambda i: (0, 0))
    rspec = pl.BlockSpec((RT, H), lambda i: (i, 0))
    return pl.pallas_call(
        _update_body,
        grid=(NT_N,),
        in_specs=[
            rspec,
            pl.BlockSpec((2, RT, H), lambda i: (0, i, 0)),
            rspec, wspec, bspec, wspec, wspec, bspec, wspec, bspec,
            bspec, bspec, wspec, wspec,
        ],
        out_specs=[rspec, rspec, rspec],
        out_shape=[jax.ShapeDtypeStruct((NP, H), _F32)] * 3,
    )(h, s2, degb, w2m, b2m, wu1a, wu1b, bu1, wu2, bu2, lng, lnb, wsn, wdn)


def _pool_body(h_ref, bi_ref, bf_ref, wo1, bo1, wo2, bo2,
               lat_ref, gr_ref, sum_acc, cnt_acc, max_acc):
    i = pl.program_id(0)

    @pl.when(i == 0)
    def _():
        sum_acc[...] = jnp.zeros_like(sum_acc)
        cnt_acc[...] = jnp.zeros_like(cnt_acc)
        max_acc[...] = jnp.full_like(max_acc, -1e30)

    h = h_ref[...]
    br = bi_ref[0]
    onehot = (lax.broadcasted_iota(jnp.int32, (NG, RT), 0) == br).astype(_F32)
    sum_acc[...] += jnp.dot(onehot, h, preferred_element_type=_F32)
    cnt_acc[...] += jnp.broadcast_to(
        jnp.sum(onehot, axis=1, keepdims=True), (NG, H))
    bf = bf_ref[...]
    for g in range(NG):
        masked = jnp.where(bf == float(g), h, -1e30)
        mg = jnp.max(masked, axis=0, keepdims=True)
        max_acc[g:g + 1, :] = jnp.maximum(max_acc[g:g + 1, :], mg)

    @pl.when(i == NT_N - 1)
    def _():
        cnt = cnt_acc[...]
        mean = sum_acc[...] / jnp.maximum(cnt, 1.0)
        mx = jnp.where(cnt > 0.0, max_acc[...], 0.0)
        gr = jnp.concatenate([mean, mx], axis=1)
        gr_ref[...] = gr
        z = _silu(jnp.dot(gr, wo1[...], preferred_element_type=_F32) + bo1[...])
        lat_ref[...] = jnp.dot(z, wo2[...], preferred_element_type=_F32) + bo2[...]


def _pool_out(h, batch3, bf, wo1, bo1, wo2, bo2):
    return pl.pallas_call(
        _pool_body,
        grid=(NT_N,),
        in_specs=[
            pl.BlockSpec((RT, H), lambda i: (i, 0)),
            pl.BlockSpec((1, 1, RT), lambda i: (i, 0, 0)),
            pl.BlockSpec((RT, H), lambda i: (i, 0)),
            pl.BlockSpec((2 * H, H), lambda i: (0, 0)),
            pl.BlockSpec((1, H), lambda i: (0, 0)),
            pl.BlockSpec((H, H), lambda i: (0, 0)),
            pl.BlockSpec((1, H), lambda i: (0, 0)),
        ],
        out_specs=[
            pl.BlockSpec((NG, H), lambda i: (0, 0)),
            pl.BlockSpec((NG, 2 * H), lambda i: (0, 0)),
        ],
        out_shape=[
            jax.ShapeDtypeStruct((NG, H), _F32),
            jax.ShapeDtypeStruct((NG, 2 * H), _F32),
        ],
        scratch_shapes=[
            pltpu.VMEM((NG, H), _F32),
            pltpu.VMEM((NG, H), _F32),
            pltpu.VMEM((NG, H), _F32),
        ],
    )(h, batch3, bf, wo1, bo1, wo2, bo2)


@functools.lru_cache(maxsize=None)
def _sc_mesh():
    return plsc.VectorSubcoreMesh(
        core_axis_name="c", subcore_axis_name="s",
        num_cores=NC, num_subcores=NS)


def _sc_edge_body(a_hbm, b_hbm, c_hbm, src_hbm, dst_hbm, out_hbm,
                  sidx, didx, didxb, abuf, bbuf, cbuf, s_sh,
                  gsem, ssem):
    cid = lax.axis_index("c")
    sid = lax.axis_index("s")
    wid = sid * NC + cid
    base = wid * EPW

    zv = jnp.zeros((16,), _F32)

    def zrow(e, _):
        for g in range(8):
            cbuf[0][e, pl.ds(g * 16, 16)] = zv
        return 0
    lax.fori_loop(0, CH, zrow, 0)
    for j in range(RPS // CH):
        pltpu.async_copy(cbuf[0], s_sh.at[pl.ds(sid * RPS + j * CH, CH)],
                         gsem[0])
    for j in range(RPS // CH):
        pltpu.make_async_copy(
            cbuf[0], s_sh.at[pl.ds(sid * RPS + j * CH, CH)], gsem[0]).wait()
    plsc.subcore_barrier()

    def fire(lc, p, sb, wait_scatter):
        if wait_scatter:
            pltpu.make_async_copy(abuf[p], s_sh.at[didxb[p]], ssem[p]).wait()
        t = sb * SBC + lc
        pltpu.async_copy(a_hbm.at[sidx.at[pl.ds(lc * CH, CH)]], abuf[p], gsem[p])
        pltpu.async_copy(b_hbm.at[didx.at[pl.ds(lc * CH, CH)]], bbuf[p], gsem[p])
        pltpu.async_copy(c_hbm.at[pl.ds(base + t * CH, CH)], cbuf[p], gsem[p])

    def process(lc, p, sb):
        for _ in range(3):
            pltpu.make_async_copy(
                c_hbm.at[pl.ds(base + sb * SBC * CH, CH)], cbuf[p],
                gsem[p]).wait()
        for o in (0, 16, 24):
            didxb[p][pl.ds(o, 16)] = didx[pl.ds(lc * CH + o, 16)]

        def ebody(e, _):
            for g in range(8):
                sl = pl.ds(g * 16, 16)
                v = abuf[p][e, sl] + bbuf[p][e, sl] + cbuf[p][e, sl]
                abuf[p][e, sl] = v / (1.0 + jnp.exp(-v))
            return 0
        lax.fori_loop(0, CH, ebody, 0)
        pltpu.async_copy(abuf[p], s_sh.at[didxb[p]], ssem[p], add=True)

    for sb in range(NSB):
        pltpu.sync_copy(src_hbm.at[pl.ds(base + sb * SBC * CH, SBC * CH)], sidx)
        pltpu.sync_copy(dst_hbm.at[pl.ds(base + sb * SBC * CH, SBC * CH)], didx)
        w = sb > 0
        fire(0, 0, sb, w)
        fire(1, 1, sb, w)

        def pair(j, _):
            lc = 2 * j
            process(lc, 0, sb)
            fire(lc + 2, 0, sb, True)
            process(lc + 1, 1, sb)
            fire(lc + 3, 1, sb, True)
            return 0
        lax.fori_loop(0, SBC // 2 - 1, pair, 0)
        process(SBC - 2, 0, sb)
        process(SBC - 1, 1, sb)

    pltpu.make_async_copy(abuf[0], s_sh.at[didxb[0]], ssem[0]).wait()
    pltpu.make_async_copy(abuf[1], s_sh.at[didxb[1]], ssem[1]).wait()

    plsc.subcore_barrier()
    pltpu.sync_copy(s_sh.at[pl.ds(sid * RPS, RPS)],
                    out_hbm.at[cid, pl.ds(sid * RPS, RPS)])


def _sc_edge(a, b, c, src, dst):
    return pl.kernel(
        _sc_edge_body,
        out_type=jax.ShapeDtypeStruct((NC, NP, H), _F32),
        mesh=_sc_mesh(),
        scratch_types=[
            pltpu.VMEM((SBC * CH,), jnp.int32),
            pltpu.VMEM((SBC * CH,), jnp.int32),
            [pltpu.VMEM((CH,), jnp.int32)] * 2,
            [pltpu.VMEM((CH, H), _F32)] * 2,
            [pltpu.VMEM((CH, H), _F32)] * 2,
            [pltpu.VMEM((CH, H), _F32)] * 2,
            pltpu.VMEM_SHARED((NP, H), _F32),
            [pltpu.SemaphoreType.DMA] * 2,
            [pltpu.SemaphoreType.DMA] * 2,
        ],
    )(a, b, c, src, dst)


def _sc_deg_body(dst_hbm, out_hbm, didx, obuf, s_sh):
    cid = lax.axis_index("c")
    sid = lax.axis_index("s")
    wid = sid * NC + cid
    base = wid * EPW

    zv = jnp.zeros((16,), _F32)

    def zrow(e, _):
        obuf[e, pl.ds(0, 16)] = zv
        return 0
    lax.fori_loop(0, CHD, zrow, 0)
    for j in range(RPS // CHD):
        pltpu.sync_copy(obuf, s_sh.at[pl.ds(sid * RPS + j * CHD, CHD)])
    plsc.subcore_barrier()

    ones = jnp.ones((16,), _F32)

    def orow(e, _):
        obuf[e, pl.ds(0, 16)] = ones
        return 0
    lax.fori_loop(0, CHD, orow, 0)

    def chunk(t, _):
        pltpu.sync_copy(dst_hbm.at[pl.ds(base + t * CHD, CHD)], didx)
        pltpu.sync_copy(obuf, s_sh.at[didx], add=True)
        return 0
    lax.fori_loop(0, NCHD, chunk, 0)

    plsc.subcore_barrier()
    pltpu.sync_copy(s_sh.at[pl.ds(sid * RPS, RPS)],
                    out_hbm.at[cid, pl.ds(sid * RPS, RPS)])


def _sc_deg(dst):
    return pl.kernel(
        _sc_deg_body,
        out_type=jax.ShapeDtypeStruct((NC, NP, 16), _F32),
        mesh=_sc_mesh(),
        scratch_types=[
            pltpu.VMEM((CHD,), jnp.int32),
            pltpu.VMEM((CHD, 16), _F32),
            pltpu.VMEM_SHARED((NP, 16), _F32),
        ],
    )(dst)


def kernel(x, edge_index, edge_attr, batch, params):
    src = edge_index[0]
    dst = edge_index[1]

    (wn1, bn1), (wn2, bn2) = params["node_proj"]
    (we1, be1), (we2, be2) = params["edge_proj"]
    layers = params["layers"]
    (wo1, bo1), (wo2, bo2) = params["out_proj"]

    ws, wd, ms, cs, w2m, b2m = [], [], [], [], [], []
    for lp in layers:
        (w1, b1), (w2, b2) = lp["msg"]
        ws.append(w1[0:H])
        wd.append(w1[H:2 * H])
        ms.append(we2 @ w1[2 * H:3 * H])
        cs.append((be2 @ w1[2 * H:3 * H] + b1)[None, :])
        w2m.append(w2)
        b2m.append(b2[None, :])
    ms = jnp.stack(ms)
    cs = jnp.stack(cs)

    xp = jnp.pad(x, ((0, NP - N), (0, 0)))
    batch_p = jnp.concatenate(
        [batch.astype(jnp.int32), jnp.full((NP - N,), NG, jnp.int32)])
    batch3 = batch_p.reshape(NT_N, 1, RT)
    bf = jnp.broadcast_to(batch_p[:, None].astype(_F32), (NP, H))

    degp = _sc_deg(dst)
    deg = degp[0, :, 0] + degp[1, :, 0]
    degb = jnp.broadcast_to(deg[:, None], (NP, H))

    h, a, b = _node_proj(xp, wn1, bn1[None, :], wn2, bn2[None, :], ws[0], wd[0])

    c_all = _edge_terms(edge_attr, we1, be1[None, :], ms, cs)

    zw = jnp.zeros((H, H), _F32)
    for l in range(NLAYERS):
        lp = layers[l]
        (wu1, bu1), (wu2, bu2) = lp["upd"]
        lng, lnb = lp["ln"]
        s2 = _sc_edge(a, b, c_all[l], src, dst)
        wsn = ws[l + 1] if l + 1 < NLAYERS else zw
        wdn = wd[l + 1] if l + 1 < NLAYERS else zw
        h, a, b = _node_update(
            h, s2, degb, w2m[l], b2m[l],
            wu1[0:H], wu1[H:2 * H], bu1[None, :], wu2, bu2[None, :],
            lng[None, :], lnb[None, :], wsn, wdn)

    lat, gr = _pool_out(h, batch3, bf, wo1, bo1[None, :], wo2, bo2[None, :])
    return lat, gr
